# baseline (speedup 1.0000x reference)
import functools

import jax
import jax.numpy as jnp
from jax import lax
from jax.experimental import pallas as pl
from jax.experimental.pallas import tpu as pltpu
from jax.experimental.pallas import tpu_sc as plsc

T = 2048
D = 768
FF = 2048
E = 8
EP = 128
K = 2
TILE = 128
NT = (K * T) // TILE + E
XROWS = NT * TILE
NB = T // TILE
NC = 2
NS = 16
NW = NC * NS
TPW = T // NW


def _router_body(x_ref, wm_ref, wg_ref,
                 h_ref, lg_ref, pr_ref, load_ref, cnt_ref, zl_ref, lb_ref,
                 r1_ref, r2_ref, wp1_ref, wp2_ref, te_ref, tv_ref,
                 tg_ref, ne_ref, hn_ref, ne2_ref, hn2_ref, ranks_ref, oh_ref):
    x = x_ref[...]
    h = x + jnp.tanh(jnp.dot(x, wm_ref[...], preferred_element_type=jnp.float32))
    h_ref[...] = h
    wg = jnp.concatenate(
        [wg_ref[...], jnp.zeros((D, EP - E), jnp.float32)], axis=1)
    logits = jnp.dot(h, wg, preferred_element_type=jnp.float32)
    lg_ref[...] = logits[:, :E]

    lane = lax.broadcasted_iota(jnp.int32, (T, EP), 1)
    lmask = lane < E
    neg = jnp.float32(-1e30)
    lm = jnp.where(lmask, logits, neg)
    rowmax = jnp.max(lm, axis=1, keepdims=True)
    ex = jnp.where(lmask, jnp.exp(lm - rowmax), 0.0)
    den = jnp.sum(ex, axis=1, keepdims=True)
    probs = ex / den
    pr_ref[...] = probs[:, :E]

    z = rowmax + jnp.log(den)
    zl_ref[...] = jnp.reshape(jnp.sum(z * z) / jnp.float32(T), (1, 1))

    big = jnp.int32(EP + 1)
    pm = jnp.where(lmask, probs, neg)
    m1 = jnp.max(pm, axis=1, keepdims=True)
    c1 = jnp.where((pm == m1) & lmask, lane, big)
    i1 = jnp.min(c1, axis=1, keepdims=True)
    oh1 = lane == i1
    p2 = jnp.where(oh1, neg, pm)
    m2 = jnp.max(p2, axis=1, keepdims=True)
    c2 = jnp.where((p2 == m2) & lmask, lane, big)
    i2 = jnp.min(c2, axis=1, keepdims=True)
    oh2 = lane == i2

    wsum = m1 + m2
    wp1_ref[...] = (m1 / wsum) * jnp.ones((1, 16), jnp.float32)
    wp2_ref[...] = (m2 / wsum) * jnp.ones((1, 16), jnp.float32)

    cnt = jnp.sum((oh1 | oh2).astype(jnp.int32), axis=0, keepdims=True)
    cnt_ref[...] = cnt[:, :E]
    load = jnp.mean(probs, axis=0, keepdims=True)
    load_ref[...] = load[:, :E]
    frac = cnt.astype(jnp.float32) / jnp.float32(T * K)
    lb_ref[...] = jnp.reshape(jnp.float32(E) * jnp.sum(frac * load), (1, 1))

    oh_ref[...] = oh1.astype(jnp.float32) + (lane == (i2 + E)).astype(jnp.float32)
    row128 = lax.broadcasted_iota(jnp.int32, (TILE, TILE), 0)
    col128 = lax.broadcasted_iota(jnp.int32, (TILE, TILE), 1)
    ltri = (row128 > col128).astype(jnp.float32)

    def blockstep(b, carry):
        blk = oh_ref[pl.ds(b * TILE, TILE), :]
        within = jnp.dot(ltri, blk, preferred_element_type=jnp.float32)
        ranks_ref[pl.ds(b * TILE, TILE), :] = within + carry
        return carry + jnp.sum(blk, axis=0, keepdims=True)

    totals = lax.fori_loop(0, NB, blockstep, jnp.zeros((1, EP), jnp.float32))
    ranks = ranks_ref[...]

    cnt_f = cnt.astype(jnp.float32)
    ntiles = jnp.floor((cnt_f + jnp.float32(TILE - 1)) * (1.0 / TILE))
    nt8 = ntiles + jnp.zeros((8, EP), jnp.float32)
    utri = (row128 < col128).astype(jnp.float32)[:EP, :EP]
    exc = jnp.dot(nt8, utri, preferred_element_type=jnp.float32)[0:1, :]
    start_rows = exc * jnp.float32(TILE)
    cum_incl = exc + ntiles
    total_tiles = jnp.sum(ntiles)

    cum1 = jnp.sum(jnp.where(oh1, ranks, 0.0), axis=1, keepdims=True)
    cum2 = jnp.sum(jnp.where(lane == (i2 + E), ranks, 0.0), axis=1,
                   keepdims=True)
    start1 = jnp.sum(jnp.where(oh1, start_rows, 0.0), axis=1, keepdims=True)
    start2 = jnp.sum(jnp.where(oh2, start_rows, 0.0), axis=1, keepdims=True)
    base2 = jnp.sum(jnp.where(oh2, totals, 0.0), axis=1, keepdims=True)
    r1_ref[...] = (start1 + cum1).astype(jnp.int32)
    r2_ref[...] = (start2 + base2 + cum2).astype(jnp.int32)

    lane1i = lax.broadcasted_iota(jnp.int32, (1, EP), 1)
    lane1 = lane1i.astype(jnp.float32)
    te = jnp.zeros((1, EP), jnp.int32)
    for e in range(E):
        ce = jnp.sum(jnp.where(lane1i == e, cum_incl, 0.0))
        te = te + (lane1 >= ce).astype(jnp.int32)
    te_ref[...] = jnp.minimum(te, E - 1)
    tv_ref[...] = (lane1 < total_tiles).astype(jnp.int32)

    tg = jnp.full((1, EP), -1, jnp.int32)
    ne = jnp.zeros((1, EP), jnp.int32)
    rank_run = jnp.zeros((1, EP), jnp.float32)
    num_groups = jnp.float32(0)
    for e in range(E):
        ce = jnp.sum(jnp.where(lane1i == e, cnt_f, 0.0))
        se = jnp.sum(jnp.where(lane1i == e, exc, 0.0))
        present = ce > 0.0
        tg = tg + jnp.where(present & (lane1 >= se), 1, 0)
        num_groups = num_groups + jnp.where(present, 1.0, 0.0)
    ne2 = jnp.zeros((1, EP), jnp.int32)
    run = jnp.float32(0)
    for e in range(E):
        ce = jnp.sum(jnp.where(lane1i == e, cnt_f, 0.0))
        present = ce > 0.0
        run = run + jnp.where(present, 1.0, 0.0)
        rank_e = run - 1.0
        tgf = tg.astype(jnp.float32)
        ne = ne + jnp.where(present & (tgf + 1.0 == rank_e), e, 0)
        ne2 = ne2 + jnp.where(present & (tgf + 2.0 == rank_e), e, 0)
    tg_ref[...] = jnp.maximum(tg, 0)
    ne_ref[...] = ne
    ne2_ref[...] = ne2
    hn_ref[...] = ((tg + 1).astype(jnp.float32) < num_groups).astype(jnp.int32)
    hn2_ref[...] = ((tg + 2).astype(jnp.float32) < num_groups).astype(jnp.int32)


def _sc_scatter_body(h_hbm, r1_hbm, r2_hbm, xg_hbm, idx1_v, idx2_v, rows_v,
                     sem):
    wid = lax.axis_index("s") * NC + lax.axis_index("c")
    base = wid * TPW
    pltpu.sync_copy(r1_hbm.at[pl.ds(base, TPW)], idx1_v)
    pltpu.sync_copy(r2_hbm.at[pl.ds(base, TPW)], idx2_v)
    pltpu.sync_copy(h_hbm.at[pl.ds(base, TPW)], rows_v)
    c1 = pltpu.async_copy(rows_v, xg_hbm.at[idx1_v], sem)
    c2 = pltpu.async_copy(rows_v, xg_hbm.at[idx2_v], sem)
    c1.wait()
    c2.wait()


def _sc_combine_body(y_hbm, r1_hbm, r2_hbm, wp1_hbm, wp2_hbm, out_hbm,
                     idx1_v, idx2_v, buf1_v, buf2_v, w1_v, w2_v, sem):
    wid = lax.axis_index("s") * NC + lax.axis_index("c")
    base = wid * TPW
    pltpu.sync_copy(r1_hbm.at[pl.ds(base, TPW)], idx1_v)
    pltpu.sync_copy(r2_hbm.at[pl.ds(base, TPW)], idx2_v)
    pltpu.sync_copy(wp1_hbm.at[pl.ds(base, TPW)], w1_v)
    pltpu.sync_copy(wp2_hbm.at[pl.ds(base, TPW)], w2_v)
    g1 = pltpu.async_copy(y_hbm.at[idx1_v], buf1_v, sem)
    g2 = pltpu.async_copy(y_hbm.at[idx2_v], buf2_v, sem)
    g1.wait()
    g2.wait()

    def row(i, _):
        wa = w1_v[i, pl.ds(0, 16)]
        wb = w2_v[i, pl.ds(0, 16)]
        for j in range(D // 16):
            a = buf1_v[i, pl.ds(j * 16, 16)]
            b = buf2_v[i, pl.ds(j * 16, 16)]
            buf1_v[i, pl.ds(j * 16, 16)] = a * wa + b * wb
        return 0

    lax.fori_loop(0, TPW, row, 0)
    pltpu.sync_copy(buf1_v, out_hbm.at[pl.ds(base, TPW)])


def _ffn_body(te_ref, tv_ref, tg_ref, ne_ref, hn_ref, ne2_ref, hn2_ref,
              xg_ref, w1_hbm, b1_ref, w2_hbm, b2_ref, y_ref,
              w1buf, w2buf, wsem):
    i = pl.program_id(0)
    e = te_ref[0, i]
    g = tg_ref[0, i]
    slot = lax.rem(g, 3)
    prev_g = tg_ref[0, jnp.maximum(i - 1, 0)]
    valid = tv_ref[0, i] == 1
    firstt = valid & ((i == 0) | (g != prev_g))

    def w_fetch(expert, dst_slot):
        return (pltpu.make_async_copy(w1_hbm.at[pl.ds(expert, 1)],
                                      w1buf.at[pl.ds(dst_slot, 1)],
                                      wsem.at[dst_slot]),
                pltpu.make_async_copy(w2_hbm.at[pl.ds(expert, 1)],
                                      w2buf.at[pl.ds(dst_slot, 1)],
                                      wsem.at[dst_slot]))

    @pl.when(i == 0)
    def _():
        c1, c2 = w_fetch(e, slot)
        c1.start()
        c2.start()

        @pl.when(hn_ref[0, i] == 1)
        def _():
            n1, n2 = w_fetch(ne_ref[0, i], lax.rem(g + 1, 3))
            n1.start()
            n2.start()

    @pl.when(firstt)
    def _():
        c1, c2 = w_fetch(e, slot)
        c1.wait()
        c2.wait()

        @pl.when(hn2_ref[0, i] == 1)
        def _():
            n1, n2 = w_fetch(ne2_ref[0, i], lax.rem(g + 2, 3))
            n1.start()
            n2.start()

    @pl.when(valid)
    def _():
        x = xg_ref[...].astype(jnp.bfloat16)
        w1 = w1buf[pl.ds(slot, 1)].reshape(D, FF).astype(jnp.bfloat16)
        hid = jax.nn.gelu(
            jnp.dot(x, w1, preferred_element_type=jnp.float32) + b1_ref[0])
        w2 = w2buf[pl.ds(slot, 1)].reshape(FF, D).astype(jnp.bfloat16)
        y_ref[...] = jnp.dot(hid.astype(jnp.bfloat16), w2,
                             preferred_element_type=jnp.float32) + b2_ref[0]


@jax.jit
def kernel(h_t, W_mix, W_gate, W1, b1, W2, b2):
    x = h_t.reshape(T, D)

    router = pl.pallas_call(
        _router_body,
        out_shape=[
            jax.ShapeDtypeStruct((T, D), jnp.float32),
            jax.ShapeDtypeStruct((T, E), jnp.float32),
            jax.ShapeDtypeStruct((T, E), jnp.float32),
            jax.ShapeDtypeStruct((1, E), jnp.float32),
            jax.ShapeDtypeStruct((1, E), jnp.int32),
            jax.ShapeDtypeStruct((1, 1), jnp.float32),
            jax.ShapeDtypeStruct((1, 1), jnp.float32),
            jax.ShapeDtypeStruct((T, 1), jnp.int32),
            jax.ShapeDtypeStruct((T, 1), jnp.int32),
            jax.ShapeDtypeStruct((T, 16), jnp.float32),
            jax.ShapeDtypeStruct((T, 16), jnp.float32),
            jax.ShapeDtypeStruct((1, EP), jnp.int32),
            jax.ShapeDtypeStruct((1, EP), jnp.int32),
            jax.ShapeDtypeStruct((1, EP), jnp.int32),
            jax.ShapeDtypeStruct((1, EP), jnp.int32),
            jax.ShapeDtypeStruct((1, EP), jnp.int32),
            jax.ShapeDtypeStruct((1, EP), jnp.int32),
            jax.ShapeDtypeStruct((1, EP), jnp.int32),
        ],
        scratch_shapes=[pltpu.VMEM((T, EP), jnp.float32),
                        pltpu.VMEM((T, EP), jnp.float32)],
    )
    (h, lg, pr, load, cnt, zl, lb, r1, r2, wp1, wp2, te, tv, tg, ne, hn,
     ne2, hn2) = router(x, W_mix, W_gate)
    r1f = r1.reshape(T)
    r2f = r2.reshape(T)

    scatter = pl.kernel(
        _sc_scatter_body,
        out_type=jax.ShapeDtypeStruct((XROWS, D), jnp.float32),
        mesh=plsc.VectorSubcoreMesh(core_axis_name="c", subcore_axis_name="s", num_cores=NC, num_subcores=NS),
        scratch_types=[
            pltpu.VMEM((TPW,), jnp.int32),
            pltpu.VMEM((TPW,), jnp.int32),
            pltpu.VMEM((TPW, D), jnp.float32),
            pltpu.SemaphoreType.DMA,
        ],
    )
    xg = scatter(h, r1f, r2f)

    ffn = pl.pallas_call(
        _ffn_body,
        grid_spec=pltpu.PrefetchScalarGridSpec(
            num_scalar_prefetch=7,
            grid=(NT,),
            in_specs=[
                pl.BlockSpec((TILE, D), lambda i, *_: (i, 0)),
                pl.BlockSpec(memory_space=pl.ANY),
                pl.BlockSpec((1, 1, FF), lambda i, te, *_: (te[0, i], 0, 0)),
                pl.BlockSpec(memory_space=pl.ANY),
                pl.BlockSpec((1, 1, D), lambda i, te, *_: (te[0, i], 0, 0)),
            ],
            out_specs=pl.BlockSpec((TILE, D), lambda i, *_: (i, 0)),
            scratch_shapes=[
                pltpu.VMEM((3, D, FF), jnp.float32),
                pltpu.VMEM((3, FF, D), jnp.float32),
                pltpu.SemaphoreType.DMA((3,)),
            ],
        ),
        out_shape=jax.ShapeDtypeStruct((XROWS, D), jnp.float32),
        compiler_params=pltpu.CompilerParams(
            dimension_semantics=("arbitrary",)),
    )
    y = ffn(te, tv, tg, ne, hn, ne2, hn2, xg, W1, b1.reshape(E, 1, FF),
            W2, b2.reshape(E, 1, D))

    combine = pl.kernel(
        _sc_combine_body,
        out_type=jax.ShapeDtypeStruct((T, D), jnp.float32),
        mesh=plsc.VectorSubcoreMesh(core_axis_name="c", subcore_axis_name="s", num_cores=NC, num_subcores=NS),
        scratch_types=[
            pltpu.VMEM((TPW,), jnp.int32),
            pltpu.VMEM((TPW,), jnp.int32),
            pltpu.VMEM((TPW, D), jnp.float32),
            pltpu.VMEM((TPW, D), jnp.float32),
            pltpu.VMEM((TPW, 16), jnp.float32),
            pltpu.VMEM((TPW, 16), jnp.float32),
            pltpu.SemaphoreType.DMA,
        ],
    )
    out = combine(y, r1f, r2f, wp1, wp2)

    return (lg, pr, out.reshape(1, T, D),
            zl.reshape(()), lb.reshape(()),
            load.reshape(E), cnt.reshape(E))


# speedup vs baseline: 1.1365x; 1.1365x over previous
import functools

import jax
import jax.numpy as jnp
from jax import lax
from jax.experimental import pallas as pl
from jax.experimental.pallas import tpu as pltpu
from jax.experimental.pallas import tpu_sc as plsc

T = 2048
D = 768
FF = 2048
E = 8
EP = 128
K = 2
TILE = 256
NT = (K * T) // TILE + E
XROWS = NT * TILE
RB = 128
NB = T // RB
NC = 2
NS = 16
NW = NC * NS
TPW = T // NW


def _router_body(x_ref, wm_ref, wg_ref,
                 h_ref, lg_ref, pr_ref, load_ref, cnt_ref, zl_ref, lb_ref,
                 r1_ref, r2_ref, wp1_ref, wp2_ref, te_ref, tv_ref,
                 tg_ref, ne_ref, hn_ref, ne2_ref, hn2_ref, ranks_ref, oh_ref):
    x = x_ref[...]
    h = x + jnp.tanh(jnp.dot(x, wm_ref[...], preferred_element_type=jnp.float32))
    h_ref[...] = h
    wg = jnp.concatenate(
        [wg_ref[...], jnp.zeros((D, EP - E), jnp.float32)], axis=1)
    logits = jnp.dot(h, wg, preferred_element_type=jnp.float32)
    lg_ref[...] = logits[:, :E]

    lane = lax.broadcasted_iota(jnp.int32, (T, EP), 1)
    lmask = lane < E
    neg = jnp.float32(-1e30)
    lm = jnp.where(lmask, logits, neg)
    rowmax = jnp.max(lm, axis=1, keepdims=True)
    ex = jnp.where(lmask, jnp.exp(lm - rowmax), 0.0)
    den = jnp.sum(ex, axis=1, keepdims=True)
    probs = ex / den
    pr_ref[...] = probs[:, :E]

    z = rowmax + jnp.log(den)
    zl_ref[...] = jnp.reshape(jnp.sum(z * z) / jnp.float32(T), (1, 1))

    big = jnp.int32(EP + 1)
    pm = jnp.where(lmask, probs, neg)
    m1 = jnp.max(pm, axis=1, keepdims=True)
    c1 = jnp.where((pm == m1) & lmask, lane, big)
    i1 = jnp.min(c1, axis=1, keepdims=True)
    oh1 = lane == i1
    p2 = jnp.where(oh1, neg, pm)
    m2 = jnp.max(p2, axis=1, keepdims=True)
    c2 = jnp.where((p2 == m2) & lmask, lane, big)
    i2 = jnp.min(c2, axis=1, keepdims=True)
    oh2 = lane == i2

    wsum = m1 + m2
    wp1_ref[...] = (m1 / wsum) * jnp.ones((1, 16), jnp.float32)
    wp2_ref[...] = (m2 / wsum) * jnp.ones((1, 16), jnp.float32)

    cnt = jnp.sum((oh1 | oh2).astype(jnp.int32), axis=0, keepdims=True)
    cnt_ref[...] = cnt[:, :E]
    load = jnp.mean(probs, axis=0, keepdims=True)
    load_ref[...] = load[:, :E]
    frac = cnt.astype(jnp.float32) / jnp.float32(T * K)
    lb_ref[...] = jnp.reshape(jnp.float32(E) * jnp.sum(frac * load), (1, 1))

    oh_ref[...] = oh1.astype(jnp.float32) + (lane == (i2 + E)).astype(jnp.float32)
    row128 = lax.broadcasted_iota(jnp.int32, (RB, RB), 0)
    col128 = lax.broadcasted_iota(jnp.int32, (RB, RB), 1)
    ltri = (row128 > col128).astype(jnp.float32)

    def blockstep(b, carry):
        blk = oh_ref[pl.ds(b * RB, RB), :]
        within = jnp.dot(ltri, blk, preferred_element_type=jnp.float32)
        ranks_ref[pl.ds(b * RB, RB), :] = within + carry
        return carry + jnp.sum(blk, axis=0, keepdims=True)

    totals = lax.fori_loop(0, NB, blockstep, jnp.zeros((1, EP), jnp.float32))
    ranks = ranks_ref[...]

    cnt_f = cnt.astype(jnp.float32)
    ntiles = jnp.floor((cnt_f + jnp.float32(TILE - 1)) * (1.0 / TILE))
    nt8 = ntiles + jnp.zeros((8, EP), jnp.float32)
    utri = (row128 < col128).astype(jnp.float32)[:EP, :EP]
    exc = jnp.dot(nt8, utri, preferred_element_type=jnp.float32)[0:1, :]
    start_rows = exc * jnp.float32(TILE)
    cum_incl = exc + ntiles
    total_tiles = jnp.sum(ntiles)

    cum1 = jnp.sum(jnp.where(oh1, ranks, 0.0), axis=1, keepdims=True)
    cum2 = jnp.sum(jnp.where(lane == (i2 + E), ranks, 0.0), axis=1,
                   keepdims=True)
    start1 = jnp.sum(jnp.where(oh1, start_rows, 0.0), axis=1, keepdims=True)
    start2 = jnp.sum(jnp.where(oh2, start_rows, 0.0), axis=1, keepdims=True)
    base2 = jnp.sum(jnp.where(oh2, totals, 0.0), axis=1, keepdims=True)
    r1_ref[...] = (start1 + cum1).astype(jnp.int32)
    r2_ref[...] = (start2 + base2 + cum2).astype(jnp.int32)

    lane1i = lax.broadcasted_iota(jnp.int32, (1, EP), 1)
    lane1 = lane1i.astype(jnp.float32)
    te = jnp.zeros((1, EP), jnp.int32)
    for e in range(E):
        ce = jnp.sum(jnp.where(lane1i == e, cum_incl, 0.0))
        te = te + (lane1 >= ce).astype(jnp.int32)
    te_ref[...] = jnp.minimum(te, E - 1)
    tv_ref[...] = (lane1 < total_tiles).astype(jnp.int32)

    tg = jnp.full((1, EP), -1, jnp.int32)
    ne = jnp.zeros((1, EP), jnp.int32)
    rank_run = jnp.zeros((1, EP), jnp.float32)
    num_groups = jnp.float32(0)
    for e in range(E):
        ce = jnp.sum(jnp.where(lane1i == e, cnt_f, 0.0))
        se = jnp.sum(jnp.where(lane1i == e, exc, 0.0))
        present = ce > 0.0
        tg = tg + jnp.where(present & (lane1 >= se), 1, 0)
        num_groups = num_groups + jnp.where(present, 1.0, 0.0)
    ne2 = jnp.zeros((1, EP), jnp.int32)
    run = jnp.float32(0)
    for e in range(E):
        ce = jnp.sum(jnp.where(lane1i == e, cnt_f, 0.0))
        present = ce > 0.0
        run = run + jnp.where(present, 1.0, 0.0)
        rank_e = run - 1.0
        tgf = tg.astype(jnp.float32)
        ne = ne + jnp.where(present & (tgf + 1.0 == rank_e), e, 0)
        ne2 = ne2 + jnp.where(present & (tgf + 2.0 == rank_e), e, 0)
    tg_ref[...] = jnp.maximum(tg, 0)
    ne_ref[...] = ne
    ne2_ref[...] = ne2
    hn_ref[...] = ((tg + 1).astype(jnp.float32) < num_groups).astype(jnp.int32)
    hn2_ref[...] = ((tg + 2).astype(jnp.float32) < num_groups).astype(jnp.int32)


def _sc_scatter_body(h_hbm, r1_hbm, r2_hbm, xg_hbm, idx1_v, idx2_v, rows_v,
                     sem):
    wid = lax.axis_index("s") * NC + lax.axis_index("c")
    base = wid * TPW
    pltpu.sync_copy(r1_hbm.at[pl.ds(base, TPW)], idx1_v)
    pltpu.sync_copy(r2_hbm.at[pl.ds(base, TPW)], idx2_v)
    pltpu.sync_copy(h_hbm.at[pl.ds(base, TPW)], rows_v)
    c1 = pltpu.async_copy(rows_v, xg_hbm.at[idx1_v], sem)
    c2 = pltpu.async_copy(rows_v, xg_hbm.at[idx2_v], sem)
    c1.wait()
    c2.wait()


def _sc_combine_body(y_hbm, r1_hbm, r2_hbm, wp1_hbm, wp2_hbm, out_hbm,
                     idx1_v, idx2_v, buf1_v, buf2_v, w1_v, w2_v, sem):
    wid = lax.axis_index("s") * NC + lax.axis_index("c")
    base = wid * TPW
    pltpu.sync_copy(r1_hbm.at[pl.ds(base, TPW)], idx1_v)
    pltpu.sync_copy(r2_hbm.at[pl.ds(base, TPW)], idx2_v)
    pltpu.sync_copy(wp1_hbm.at[pl.ds(base, TPW)], w1_v)
    pltpu.sync_copy(wp2_hbm.at[pl.ds(base, TPW)], w2_v)
    g1 = pltpu.async_copy(y_hbm.at[idx1_v], buf1_v, sem)
    g2 = pltpu.async_copy(y_hbm.at[idx2_v], buf2_v, sem)
    g1.wait()
    g2.wait()

    def row(i, _):
        wa = w1_v[i, pl.ds(0, 16)]
        wb = w2_v[i, pl.ds(0, 16)]
        for j in range(D // 16):
            a = buf1_v[i, pl.ds(j * 16, 16)]
            b = buf2_v[i, pl.ds(j * 16, 16)]
            buf1_v[i, pl.ds(j * 16, 16)] = a * wa + b * wb
        return 0

    lax.fori_loop(0, TPW, row, 0)
    pltpu.sync_copy(buf1_v, out_hbm.at[pl.ds(base, TPW)])


def _ffn_body(te_ref, tv_ref, tg_ref, ne_ref, hn_ref, ne2_ref, hn2_ref,
              xg_ref, w1_hbm, b1_ref, w2_hbm, b2_ref, y_ref,
              w1buf, w2buf, wsem):
    i = pl.program_id(0)
    e = te_ref[0, i]
    g = tg_ref[0, i]
    slot = lax.rem(g, 3)
    prev_g = tg_ref[0, jnp.maximum(i - 1, 0)]
    valid = tv_ref[0, i] == 1
    firstt = valid & ((i == 0) | (g != prev_g))

    def w_fetch(expert, dst_slot):
        return (pltpu.make_async_copy(w1_hbm.at[pl.ds(expert, 1)],
                                      w1buf.at[pl.ds(dst_slot, 1)],
                                      wsem.at[dst_slot]),
                pltpu.make_async_copy(w2_hbm.at[pl.ds(expert, 1)],
                                      w2buf.at[pl.ds(dst_slot, 1)],
                                      wsem.at[dst_slot]))

    @pl.when(i == 0)
    def _():
        c1, c2 = w_fetch(e, slot)
        c1.start()
        c2.start()

        @pl.when(hn_ref[0, i] == 1)
        def _():
            n1, n2 = w_fetch(ne_ref[0, i], lax.rem(g + 1, 3))
            n1.start()
            n2.start()

    @pl.when(firstt)
    def _():
        c1, c2 = w_fetch(e, slot)
        c1.wait()
        c2.wait()

        @pl.when(hn2_ref[0, i] == 1)
        def _():
            n1, n2 = w_fetch(ne2_ref[0, i], lax.rem(g + 2, 3))
            n1.start()
            n2.start()

    @pl.when(valid)
    def _():
        x = xg_ref[...].astype(jnp.bfloat16)
        w1 = w1buf[pl.ds(slot, 1)].reshape(D, FF).astype(jnp.bfloat16)
        hid = jax.nn.gelu(
            jnp.dot(x, w1, preferred_element_type=jnp.float32) + b1_ref[0])
        w2 = w2buf[pl.ds(slot, 1)].reshape(FF, D).astype(jnp.bfloat16)
        y_ref[...] = jnp.dot(hid.astype(jnp.bfloat16), w2,
                             preferred_element_type=jnp.float32) + b2_ref[0]


@jax.jit
def kernel(h_t, W_mix, W_gate, W1, b1, W2, b2):
    x = h_t.reshape(T, D)

    router = pl.pallas_call(
        _router_body,
        out_shape=[
            jax.ShapeDtypeStruct((T, D), jnp.float32),
            jax.ShapeDtypeStruct((T, E), jnp.float32),
            jax.ShapeDtypeStruct((T, E), jnp.float32),
            jax.ShapeDtypeStruct((1, E), jnp.float32),
            jax.ShapeDtypeStruct((1, E), jnp.int32),
            jax.ShapeDtypeStruct((1, 1), jnp.float32),
            jax.ShapeDtypeStruct((1, 1), jnp.float32),
            jax.ShapeDtypeStruct((T, 1), jnp.int32),
            jax.ShapeDtypeStruct((T, 1), jnp.int32),
            jax.ShapeDtypeStruct((T, 16), jnp.float32),
            jax.ShapeDtypeStruct((T, 16), jnp.float32),
            jax.ShapeDtypeStruct((1, EP), jnp.int32),
            jax.ShapeDtypeStruct((1, EP), jnp.int32),
            jax.ShapeDtypeStruct((1, EP), jnp.int32),
            jax.ShapeDtypeStruct((1, EP), jnp.int32),
            jax.ShapeDtypeStruct((1, EP), jnp.int32),
            jax.ShapeDtypeStruct((1, EP), jnp.int32),
            jax.ShapeDtypeStruct((1, EP), jnp.int32),
        ],
        scratch_shapes=[pltpu.VMEM((T, EP), jnp.float32),
                        pltpu.VMEM((T, EP), jnp.float32)],
    )
    (h, lg, pr, load, cnt, zl, lb, r1, r2, wp1, wp2, te, tv, tg, ne, hn,
     ne2, hn2) = router(x, W_mix, W_gate)
    r1f = r1.reshape(T)
    r2f = r2.reshape(T)

    scatter = pl.kernel(
        _sc_scatter_body,
        out_type=jax.ShapeDtypeStruct((XROWS, D), jnp.float32),
        mesh=plsc.VectorSubcoreMesh(core_axis_name="c", subcore_axis_name="s", num_cores=NC, num_subcores=NS),
        scratch_types=[
            pltpu.VMEM((TPW,), jnp.int32),
            pltpu.VMEM((TPW,), jnp.int32),
            pltpu.VMEM((TPW, D), jnp.float32),
            pltpu.SemaphoreType.DMA,
        ],
    )
    xg = scatter(h, r1f, r2f)

    ffn = pl.pallas_call(
        _ffn_body,
        grid_spec=pltpu.PrefetchScalarGridSpec(
            num_scalar_prefetch=7,
            grid=(NT,),
            in_specs=[
                pl.BlockSpec((TILE, D), lambda i, *_: (i, 0)),
                pl.BlockSpec(memory_space=pl.ANY),
                pl.BlockSpec((1, 1, FF), lambda i, te, *_: (te[0, i], 0, 0)),
                pl.BlockSpec(memory_space=pl.ANY),
                pl.BlockSpec((1, 1, D), lambda i, te, *_: (te[0, i], 0, 0)),
            ],
            out_specs=pl.BlockSpec((TILE, D), lambda i, *_: (i, 0)),
            scratch_shapes=[
                pltpu.VMEM((3, D, FF), jnp.float32),
                pltpu.VMEM((3, FF, D), jnp.float32),
                pltpu.SemaphoreType.DMA((3,)),
            ],
        ),
        out_shape=jax.ShapeDtypeStruct((XROWS, D), jnp.float32),
        compiler_params=pltpu.CompilerParams(
            dimension_semantics=("arbitrary",)),
    )
    y = ffn(te, tv, tg, ne, hn, ne2, hn2, xg, W1, b1.reshape(E, 1, FF),
            W2, b2.reshape(E, 1, D))

    combine = pl.kernel(
        _sc_combine_body,
        out_type=jax.ShapeDtypeStruct((T, D), jnp.float32),
        mesh=plsc.VectorSubcoreMesh(core_axis_name="c", subcore_axis_name="s", num_cores=NC, num_subcores=NS),
        scratch_types=[
            pltpu.VMEM((TPW,), jnp.int32),
            pltpu.VMEM((TPW,), jnp.int32),
            pltpu.VMEM((TPW, D), jnp.float32),
            pltpu.VMEM((TPW, D), jnp.float32),
            pltpu.VMEM((TPW, 16), jnp.float32),
            pltpu.VMEM((TPW, 16), jnp.float32),
            pltpu.SemaphoreType.DMA,
        ],
    )
    out = combine(y, r1f, r2f, wp1, wp2)

    return (lg, pr, out.reshape(1, T, D),
            zl.reshape(()), lb.reshape(()),
            load.reshape(E), cnt.reshape(E))


# speedup vs baseline: 1.1635x; 1.0238x over previous
import functools

import jax
import jax.numpy as jnp
from jax import lax
from jax.experimental import pallas as pl
from jax.experimental.pallas import tpu as pltpu
from jax.experimental.pallas import tpu_sc as plsc

T = 2048
D = 768
FF = 2048
E = 8
EP = 128
K = 2
TILE = 256
NT = (K * T) // TILE + E
XROWS = NT * TILE
RB = 128
NB = T // RB
NC = 2
NS = 16
NW = NC * NS
TPW = T // NW


def _router_body(xt_ref, wm_ref, wg_ref,
                 h_ref, lg_ref, pr_ref, load_ref, cnt_ref, zl_ref, lb_ref,
                 r1_ref, r2_ref, wp1_ref, wp2_ref, te_ref, tv_ref,
                 tg_ref, ne_ref, hn_ref, ne2_ref, hn2_ref, ranks_ref, oh_ref):
    x = xt_ref[0]
    h = x + jnp.tanh(jnp.dot(x, wm_ref[...], preferred_element_type=jnp.float32))
    h_ref[...] = h
    wg = jnp.concatenate(
        [wg_ref[...], jnp.zeros((D, EP - E), jnp.float32)], axis=1)
    logits = jnp.dot(h, wg, preferred_element_type=jnp.float32)
    lg_ref[...] = logits[:, :E]

    lane = lax.broadcasted_iota(jnp.int32, (T, EP), 1)
    lmask = lane < E
    neg = jnp.float32(-1e30)
    lm = jnp.where(lmask, logits, neg)
    rowmax = jnp.max(lm, axis=1, keepdims=True)
    ex = jnp.where(lmask, jnp.exp(lm - rowmax), 0.0)
    den = jnp.sum(ex, axis=1, keepdims=True)
    probs = ex / den
    pr_ref[...] = probs[:, :E]

    z = rowmax + jnp.log(den)
    zl_ref[...] = jnp.reshape(jnp.sum(z * z) / jnp.float32(T), (1, 1))

    big = jnp.int32(EP + 1)
    pm = jnp.where(lmask, probs, neg)
    m1 = jnp.max(pm, axis=1, keepdims=True)
    c1 = jnp.where((pm == m1) & lmask, lane, big)
    i1 = jnp.min(c1, axis=1, keepdims=True)
    oh1 = lane == i1
    p2 = jnp.where(oh1, neg, pm)
    m2 = jnp.max(p2, axis=1, keepdims=True)
    c2 = jnp.where((p2 == m2) & lmask, lane, big)
    i2 = jnp.min(c2, axis=1, keepdims=True)
    oh2 = lane == i2

    wsum = m1 + m2
    wp1_ref[...] = (m1 / wsum) * jnp.ones((1, EP), jnp.float32)
    wp2_ref[...] = (m2 / wsum) * jnp.ones((1, EP), jnp.float32)

    cnt = jnp.sum((oh1 | oh2).astype(jnp.int32), axis=0, keepdims=True)
    cnt_ref[...] = cnt[:, :E]
    load = jnp.mean(probs, axis=0, keepdims=True)
    load_ref[...] = load[:, :E]
    frac = cnt.astype(jnp.float32) / jnp.float32(T * K)
    lb_ref[...] = jnp.reshape(jnp.float32(E) * jnp.sum(frac * load), (1, 1))

    oh_ref[...] = oh1.astype(jnp.float32) + (lane == (i2 + E)).astype(jnp.float32)
    row128 = lax.broadcasted_iota(jnp.int32, (RB, RB), 0)
    col128 = lax.broadcasted_iota(jnp.int32, (RB, RB), 1)
    ltri = (row128 > col128).astype(jnp.float32)

    def blockstep(b, carry):
        blk = oh_ref[pl.ds(b * RB, RB), :]
        within = jnp.dot(ltri, blk, preferred_element_type=jnp.float32)
        ranks_ref[pl.ds(b * RB, RB), :] = within + carry
        return carry + jnp.sum(blk, axis=0, keepdims=True)

    totals = lax.fori_loop(0, NB, blockstep, jnp.zeros((1, EP), jnp.float32))
    ranks = ranks_ref[...]

    cnt_f = cnt.astype(jnp.float32)
    ntiles = jnp.floor((cnt_f + jnp.float32(TILE - 1)) * (1.0 / TILE))
    nt8 = ntiles + jnp.zeros((8, EP), jnp.float32)
    utri = (row128 < col128).astype(jnp.float32)[:EP, :EP]
    exc = jnp.dot(nt8, utri, preferred_element_type=jnp.float32)[0:1, :]
    start_rows = exc * jnp.float32(TILE)
    cum_incl = exc + ntiles
    total_tiles = jnp.sum(ntiles)

    cum1 = jnp.sum(jnp.where(oh1, ranks, 0.0), axis=1, keepdims=True)
    cum2 = jnp.sum(jnp.where(lane == (i2 + E), ranks, 0.0), axis=1,
                   keepdims=True)
    start1 = jnp.sum(jnp.where(oh1, start_rows, 0.0), axis=1, keepdims=True)
    start2 = jnp.sum(jnp.where(oh2, start_rows, 0.0), axis=1, keepdims=True)
    base2 = jnp.sum(jnp.where(oh2, totals, 0.0), axis=1, keepdims=True)
    r1_ref[...] = jnp.reshape((start1 + cum1).astype(jnp.int32), (T,))
    r2_ref[...] = jnp.reshape((start2 + base2 + cum2).astype(jnp.int32), (T,))

    lane1i = lax.broadcasted_iota(jnp.int32, (1, EP), 1)
    lane1 = lane1i.astype(jnp.float32)
    te = jnp.zeros((1, EP), jnp.int32)
    for e in range(E):
        ce = jnp.sum(jnp.where(lane1i == e, cum_incl, 0.0))
        te = te + (lane1 >= ce).astype(jnp.int32)
    te_ref[...] = jnp.minimum(te, E - 1)
    tv_ref[...] = (lane1 < total_tiles).astype(jnp.int32)

    tg = jnp.full((1, EP), -1, jnp.int32)
    ne = jnp.zeros((1, EP), jnp.int32)
    rank_run = jnp.zeros((1, EP), jnp.float32)
    num_groups = jnp.float32(0)
    for e in range(E):
        ce = jnp.sum(jnp.where(lane1i == e, cnt_f, 0.0))
        se = jnp.sum(jnp.where(lane1i == e, exc, 0.0))
        present = ce > 0.0
        tg = tg + jnp.where(present & (lane1 >= se), 1, 0)
        num_groups = num_groups + jnp.where(present, 1.0, 0.0)
    ne2 = jnp.zeros((1, EP), jnp.int32)
    run = jnp.float32(0)
    for e in range(E):
        ce = jnp.sum(jnp.where(lane1i == e, cnt_f, 0.0))
        present = ce > 0.0
        run = run + jnp.where(present, 1.0, 0.0)
        rank_e = run - 1.0
        tgf = tg.astype(jnp.float32)
        ne = ne + jnp.where(present & (tgf + 1.0 == rank_e), e, 0)
        ne2 = ne2 + jnp.where(present & (tgf + 2.0 == rank_e), e, 0)
    tg_ref[...] = jnp.maximum(tg, 0)
    ne_ref[...] = ne
    ne2_ref[...] = ne2
    hn_ref[...] = ((tg + 1).astype(jnp.float32) < num_groups).astype(jnp.int32)
    hn2_ref[...] = ((tg + 2).astype(jnp.float32) < num_groups).astype(jnp.int32)


def _sc_scatter_body(h_hbm, r1_hbm, r2_hbm, xg_hbm, idx1_v, idx2_v, rows_v,
                     sem):
    wid = lax.axis_index("s") * NC + lax.axis_index("c")
    base = wid * TPW
    pltpu.sync_copy(r1_hbm.at[pl.ds(base, TPW)], idx1_v)
    pltpu.sync_copy(r2_hbm.at[pl.ds(base, TPW)], idx2_v)
    pltpu.sync_copy(h_hbm.at[pl.ds(base, TPW)], rows_v)
    c1 = pltpu.async_copy(rows_v, xg_hbm.at[idx1_v], sem)
    c2 = pltpu.async_copy(rows_v, xg_hbm.at[idx2_v], sem)
    c1.wait()
    c2.wait()


def _sc_combine_body(y_hbm, r1_hbm, r2_hbm, wp1_hbm, wp2_hbm, out_hbm,
                     idx1_v, idx2_v, buf1_v, buf2_v, w1_v, w2_v, sem):
    wid = lax.axis_index("s") * NC + lax.axis_index("c")
    base = wid * TPW
    pltpu.sync_copy(r1_hbm.at[pl.ds(base, TPW)], idx1_v)
    pltpu.sync_copy(r2_hbm.at[pl.ds(base, TPW)], idx2_v)
    pltpu.sync_copy(wp1_hbm.at[pl.ds(base, TPW)], w1_v)
    pltpu.sync_copy(wp2_hbm.at[pl.ds(base, TPW)], w2_v)
    g1 = pltpu.async_copy(y_hbm.at[idx1_v], buf1_v, sem)
    g2 = pltpu.async_copy(y_hbm.at[idx2_v], buf2_v, sem)
    g1.wait()
    g2.wait()

    def row(i, _):
        wa = w1_v[i, pl.ds(0, 16)]
        wb = w2_v[i, pl.ds(0, 16)]
        for j in range(D // 16):
            a = buf1_v[i, pl.ds(j * 16, 16)]
            b = buf2_v[i, pl.ds(j * 16, 16)]
            buf1_v[i, pl.ds(j * 16, 16)] = a * wa + b * wb
        return 0

    lax.fori_loop(0, TPW, row, 0)
    pltpu.sync_copy(buf1_v, out_hbm.at[pl.ds(base, TPW)])


def _ffn_body(te_ref, tv_ref, tg_ref, ne_ref, hn_ref, ne2_ref, hn2_ref,
              xg_ref, w1_hbm, b1_ref, w2_hbm, b2_ref, y_ref,
              w1buf, w2buf, wsem):
    i = pl.program_id(0)
    e = te_ref[0, i]
    g = tg_ref[0, i]
    slot = lax.rem(g, 3)
    prev_g = tg_ref[0, jnp.maximum(i - 1, 0)]
    valid = tv_ref[0, i] == 1
    firstt = valid & ((i == 0) | (g != prev_g))

    def w_fetch(expert, dst_slot):
        return (pltpu.make_async_copy(w1_hbm.at[pl.ds(expert, 1)],
                                      w1buf.at[pl.ds(dst_slot, 1)],
                                      wsem.at[dst_slot]),
                pltpu.make_async_copy(w2_hbm.at[pl.ds(expert, 1)],
                                      w2buf.at[pl.ds(dst_slot, 1)],
                                      wsem.at[dst_slot]))

    @pl.when(i == 0)
    def _():
        c1, c2 = w_fetch(e, slot)
        c1.start()
        c2.start()

        @pl.when(hn_ref[0, i] == 1)
        def _():
            n1, n2 = w_fetch(ne_ref[0, i], lax.rem(g + 1, 3))
            n1.start()
            n2.start()

    @pl.when(firstt)
    def _():
        c1, c2 = w_fetch(e, slot)
        c1.wait()
        c2.wait()

        @pl.when(hn2_ref[0, i] == 1)
        def _():
            n1, n2 = w_fetch(ne2_ref[0, i], lax.rem(g + 2, 3))
            n1.start()
            n2.start()

    @pl.when(valid)
    def _():
        x = xg_ref[...].astype(jnp.bfloat16)
        w1 = w1buf[pl.ds(slot, 1)].reshape(D, FF).astype(jnp.bfloat16)
        hid = jax.nn.gelu(
            jnp.dot(x, w1, preferred_element_type=jnp.float32) + b1_ref[0])
        w2 = w2buf[pl.ds(slot, 1)].reshape(FF, D).astype(jnp.bfloat16)
        y_ref[...] = jnp.dot(hid.astype(jnp.bfloat16), w2,
                             preferred_element_type=jnp.float32) + b2_ref[0]


@jax.jit
def kernel(h_t, W_mix, W_gate, W1, b1, W2, b2):

    router = pl.pallas_call(
        _router_body,
        out_shape=[
            jax.ShapeDtypeStruct((T, D), jnp.float32),
            jax.ShapeDtypeStruct((T, E), jnp.float32),
            jax.ShapeDtypeStruct((T, E), jnp.float32),
            jax.ShapeDtypeStruct((1, E), jnp.float32),
            jax.ShapeDtypeStruct((1, E), jnp.int32),
            jax.ShapeDtypeStruct((1, 1), jnp.float32),
            jax.ShapeDtypeStruct((1, 1), jnp.float32),
            jax.ShapeDtypeStruct((T,), jnp.int32),
            jax.ShapeDtypeStruct((T,), jnp.int32),
            jax.ShapeDtypeStruct((T, EP), jnp.float32),
            jax.ShapeDtypeStruct((T, EP), jnp.float32),
            jax.ShapeDtypeStruct((1, EP), jnp.int32),
            jax.ShapeDtypeStruct((1, EP), jnp.int32),
            jax.ShapeDtypeStruct((1, EP), jnp.int32),
            jax.ShapeDtypeStruct((1, EP), jnp.int32),
            jax.ShapeDtypeStruct((1, EP), jnp.int32),
            jax.ShapeDtypeStruct((1, EP), jnp.int32),
            jax.ShapeDtypeStruct((1, EP), jnp.int32),
        ],
        scratch_shapes=[pltpu.VMEM((T, EP), jnp.float32),
                        pltpu.VMEM((T, EP), jnp.float32)],
    )
    (h, lg, pr, load, cnt, zl, lb, r1f, r2f, wp1, wp2, te, tv, tg, ne, hn,
     ne2, hn2) = router(h_t, W_mix, W_gate)

    scatter = pl.kernel(
        _sc_scatter_body,
        out_type=jax.ShapeDtypeStruct((XROWS, D), jnp.float32),
        mesh=plsc.VectorSubcoreMesh(core_axis_name="c", subcore_axis_name="s", num_cores=NC, num_subcores=NS),
        scratch_types=[
            pltpu.VMEM((TPW,), jnp.int32),
            pltpu.VMEM((TPW,), jnp.int32),
            pltpu.VMEM((TPW, D), jnp.float32),
            pltpu.SemaphoreType.DMA,
        ],
    )
    xg = scatter(h, r1f, r2f)

    ffn = pl.pallas_call(
        _ffn_body,
        grid_spec=pltpu.PrefetchScalarGridSpec(
            num_scalar_prefetch=7,
            grid=(NT,),
            in_specs=[
                pl.BlockSpec((TILE, D), lambda i, *_: (i, 0)),
                pl.BlockSpec(memory_space=pl.ANY),
                pl.BlockSpec((1, 1, FF), lambda i, te, *_: (te[0, i], 0, 0)),
                pl.BlockSpec(memory_space=pl.ANY),
                pl.BlockSpec((1, 1, D), lambda i, te, *_: (te[0, i], 0, 0)),
            ],
            out_specs=pl.BlockSpec((TILE, D), lambda i, *_: (i, 0)),
            scratch_shapes=[
                pltpu.VMEM((3, D, FF), jnp.float32),
                pltpu.VMEM((3, FF, D), jnp.float32),
                pltpu.SemaphoreType.DMA((3,)),
            ],
        ),
        out_shape=jax.ShapeDtypeStruct((XROWS, D), jnp.float32),
        compiler_params=pltpu.CompilerParams(
            dimension_semantics=("arbitrary",)),
    )
    y = ffn(te, tv, tg, ne, hn, ne2, hn2, xg, W1, b1.reshape(E, 1, FF),
            W2, b2.reshape(E, 1, D))

    combine = pl.kernel(
        _sc_combine_body,
        out_type=jax.ShapeDtypeStruct((T, D), jnp.float32),
        mesh=plsc.VectorSubcoreMesh(core_axis_name="c", subcore_axis_name="s", num_cores=NC, num_subcores=NS),
        scratch_types=[
            pltpu.VMEM((TPW,), jnp.int32),
            pltpu.VMEM((TPW,), jnp.int32),
            pltpu.VMEM((TPW, D), jnp.float32),
            pltpu.VMEM((TPW, D), jnp.float32),
            pltpu.VMEM((TPW, EP), jnp.float32),
            pltpu.VMEM((TPW, EP), jnp.float32),
            pltpu.SemaphoreType.DMA,
        ],
    )
    out = combine(y, r1f, r2f, wp1, wp2)

    return (lg, pr, out.reshape(1, T, D),
            zl.reshape(()), lb.reshape(()),
            load.reshape(E), cnt.reshape(E))
